# Initial kernel scaffold; baseline (speedup 1.0000x reference)
import jax
import jax.numpy as jnp
from jax.experimental import pallas as pl

_N = 9216
_D = 256
_K = 1024
_BN = 512
_NB = _N // _BN


def _vq_kernel(z_ref, c_ref, zq_ref, idx_ref, commit_ref):
    i = pl.program_id(0)
    zb = z_ref[...]
    cb = c_ref[...]
    z_sq = jnp.sum(zb * zb, axis=1, keepdims=True)
    c_sq = jnp.sum(cb * cb, axis=1)
    m = jnp.dot(zb, cb.T, preferred_element_type=jnp.float32)
    dist = z_sq - 2.0 * m + c_sq[None, :]
    idx = jnp.argmin(dist, axis=1).astype(jnp.int32)
    oh = (idx[:, None] == jax.lax.broadcasted_iota(jnp.int32, (_BN, _K), 1))
    zq = jnp.dot(oh.astype(jnp.float32), cb,
                 preferred_element_type=jnp.float32)
    zq_ref[...] = zb + (zq - zb)
    idx_ref[0, 0, :] = idx
    diff = zb - zq
    part = jnp.sum(diff * diff)

    @pl.when(i == 0)
    def _init():
        commit_ref[0, 0] = 0.0

    commit_ref[0, 0] += part


def kernel(z, codebook):
    zq, idx3, commit_sum = pl.pallas_call(
        _vq_kernel,
        grid=(_NB,),
        in_specs=[
            pl.BlockSpec((_BN, _D), lambda i: (i, 0)),
            pl.BlockSpec((_K, _D), lambda i: (0, 0)),
        ],
        out_specs=[
            pl.BlockSpec((_BN, _D), lambda i: (i, 0)),
            pl.BlockSpec((1, 1, _BN), lambda i: (i, 0, 0)),
            pl.BlockSpec((1, 1), lambda i: (0, 0)),
        ],
        out_shape=[
            jax.ShapeDtypeStruct((_N, _D), jnp.float32),
            jax.ShapeDtypeStruct((_NB, 1, _BN), jnp.int32),
            jax.ShapeDtypeStruct((1, 1), jnp.float32),
        ],
    )(z, codebook)
    indices = idx3.reshape(_N)
    commit = commit_sum[0, 0] / (_N * _D)
    return (zq, indices, commit)


# speedup vs baseline: 1.7745x; 1.7745x over previous
import jax
import jax.numpy as jnp
from jax.experimental import pallas as pl

_N = 9216
_D = 256
_K = 1024
_BN = 512
_NB = _N // _BN


def _vq_kernel(z_ref, c_ref, zq_ref, idx_ref, commit_ref):
    i = pl.program_id(0)
    zb = z_ref[...]
    cb = c_ref[...]
    z_sq = jnp.sum(zb * zb, axis=1, keepdims=True)
    c_sq = jnp.sum(cb * cb, axis=1)
    m = jnp.dot(zb, cb.T, preferred_element_type=jnp.float32)
    dist = z_sq - 2.0 * m + c_sq[None, :]
    idx = jnp.argmin(dist, axis=1).astype(jnp.int32)
    oh = (idx[:, None] == jax.lax.broadcasted_iota(jnp.int32, (_BN, _K), 1))
    zq = jnp.dot(oh.astype(jnp.float32), cb,
                 preferred_element_type=jnp.float32)
    zq_ref[...] = zb + (zq - zb)
    idx_ref[...] = idx.reshape(1, 1, _BN)
    diff = zb - zq
    part = jnp.sum(diff * diff).reshape(1, 1)

    @pl.when(i == 0)
    def _init():
        commit_ref[...] = jnp.zeros((1, 1), jnp.float32)

    commit_ref[...] += part


def kernel(z, codebook):
    zq, idx3, commit_sum = pl.pallas_call(
        _vq_kernel,
        grid=(_NB,),
        in_specs=[
            pl.BlockSpec((_BN, _D), lambda i: (i, 0)),
            pl.BlockSpec((_K, _D), lambda i: (0, 0)),
        ],
        out_specs=[
            pl.BlockSpec((_BN, _D), lambda i: (i, 0)),
            pl.BlockSpec((1, 1, _BN), lambda i: (i, 0, 0)),
            pl.BlockSpec((1, 1), lambda i: (0, 0)),
        ],
        out_shape=[
            jax.ShapeDtypeStruct((_N, _D), jnp.float32),
            jax.ShapeDtypeStruct((_NB, 1, _BN), jnp.int32),
            jax.ShapeDtypeStruct((1, 1), jnp.float32),
        ],
    )(z, codebook)
    indices = idx3.reshape(_N)
    commit = commit_sum[0, 0] / (_N * _D)
    return (zq, indices, commit)
